# baseline (speedup 1.0000x reference)
import functools

import jax
import jax.numpy as jnp
from jax import lax
from jax.experimental import pallas as pl
from jax.experimental.pallas import tpu as pltpu
from jax.experimental.pallas import tpu_sc as plsc

_NS = 16
_NC = 2
_C = 128


def _mm3_body(h_ref, s_ref, m_ref, wh_ref, ws_ref, wm_ref, ng_ref, nf_ref,
              bm_ref, hw_ref, sw_ref, mb_ref):
    hw_ref[...] = jnp.dot(h_ref[...], wh_ref[...],
                          preferred_element_type=jnp.float32) * ng_ref[...]
    sw_ref[...] = jnp.dot(s_ref[...], ws_ref[...],
                          preferred_element_type=jnp.float32) * nf_ref[...]
    mb_ref[...] = jnp.dot(m_ref[...], wm_ref[...],
                          preferred_element_type=jnp.float32) + bm_ref[...]


def _mm3(h, s, m, wh, ws, wm, norm_g, norm_f, bm, bm_rows):
    n, d_in = h.shape
    d_out = wh.shape[1]
    grid = (n // bm_rows,)
    row_spec = pl.BlockSpec((bm_rows, d_in), lambda i: (i, 0))
    out_spec = pl.BlockSpec((bm_rows, d_out), lambda i: (i, 0))
    w_spec = pl.BlockSpec(wh.shape, lambda i: (0, 0))
    nrm_spec = pl.BlockSpec((bm_rows, 1), lambda i: (i, 0))
    b_spec = pl.BlockSpec((1, d_out), lambda i: (0, 0))
    out_shape = jax.ShapeDtypeStruct((n, d_out), jnp.float32)
    return pl.pallas_call(
        _mm3_body,
        grid=grid,
        in_specs=[row_spec, row_spec, row_spec, w_spec, w_spec, w_spec,
                  nrm_spec, nrm_spec, b_spec],
        out_specs=[out_spec, out_spec, out_spec],
        out_shape=[out_shape, out_shape, out_shape],
    )(h, s, m, wh, ws, wm, norm_g, norm_f, bm.reshape(1, d_out))


def _sc_aggregate(hw, sw, srcg, dstg, srcf, dstf, zeros, npad, ept):
    d = hw.shape[1]
    rows_pt = npad // _NS
    mesh = plsc.VectorSubcoreMesh(core_axis_name="c", subcore_axis_name="s")
    out_t = jax.ShapeDtypeStruct((npad, d), jnp.float32)

    nbuf = 3
    @functools.partial(
        pl.kernel,
        out_type=[out_t, out_t],
        mesh=mesh,
        scratch_types=[
            [pltpu.VMEM((_C,), jnp.int32) for _ in range(nbuf)],
            [pltpu.VMEM((_C,), jnp.int32) for _ in range(nbuf)],
            [pltpu.VMEM((_C, d), jnp.float32) for _ in range(nbuf)],
            pltpu.VMEM_SHARED((npad, d), jnp.float32),
            [pltpu.SemaphoreType.DMA for _ in range(nbuf)],
            [pltpu.SemaphoreType.DMA for _ in range(nbuf)],
            [pltpu.SemaphoreType.DMA for _ in range(nbuf)],
        ],
    )
    def agg(hw_h, sw_h, srcg_h, dstg_h, srcf_h, dstf_h, zeros_h,
            outg_h, outf_h, idx_s, idx_d, rows, acc, sem_i, sem_g, sem_s):
        c = lax.axis_index("c")
        s = lax.axis_index("s")
        r0 = s * rows_pt
        pltpu.sync_copy(zeros_h.at[pl.ds(r0, rows_pt)],
                        acc.at[pl.ds(r0, rows_pt)])
        plsc.subcore_barrier()

        ebase = s * ept

        def edge_loop(table_h, src_h, dst_h):
            def front(i, j, drain):
                base = ebase + i * _C
                if drain:
                    pltpu.make_async_copy(
                        rows[j], acc.at[idx_d[j]], sem_s[j]).wait()
                pltpu.async_copy(src_h.at[pl.ds(base, _C)], idx_s[j],
                                 sem_i[j])
                pltpu.async_copy(dst_h.at[pl.ds(base, _C)], idx_d[j],
                                 sem_i[j])
                pltpu.make_async_copy(src_h.at[pl.ds(base, _C)], idx_s[j],
                                      sem_i[j]).wait()
                pltpu.make_async_copy(dst_h.at[pl.ds(base, _C)], idx_d[j],
                                      sem_i[j]).wait()
                pltpu.async_copy(table_h.at[idx_s[j]], rows[j], sem_g[j])

            def back(j):
                pltpu.make_async_copy(table_h.at[idx_s[j]], rows[j],
                                      sem_g[j]).wait()
                pltpu.async_copy(rows[j], acc.at[idx_d[j]], sem_s[j],
                                 add=True)

            for j in range(nbuf):
                front(j, j, False)
                if j:
                    back(j - 1)

            def body(b, carry):
                i = nbuf + b * nbuf
                for j in range(nbuf):
                    back((j + nbuf - 1) % nbuf)
                    front(i + j, j, True)
                return carry
            lax.fori_loop(0, ept // _C // nbuf - 1, body, 0)
            back(nbuf - 1)
            for j in range(nbuf):
                pltpu.make_async_copy(rows[j], acc.at[idx_d[j]],
                                      sem_s[j]).wait()

        @pl.when(c == 0)
        def _():
            edge_loop(hw_h, srcg_h, dstg_h)

        @pl.when(c == 1)
        def _():
            edge_loop(sw_h, srcf_h, dstf_h)

        plsc.subcore_barrier()

        @pl.when(c == 0)
        def _():
            pltpu.sync_copy(acc.at[pl.ds(r0, rows_pt)],
                            outg_h.at[pl.ds(r0, rows_pt)])

        @pl.when(c == 1)
        def _():
            pltpu.sync_copy(acc.at[pl.ds(r0, rows_pt)],
                            outf_h.at[pl.ds(r0, rows_pt)])

    return agg(hw, sw, srcg, dstg, srcf, dstf, zeros)


def _fuse_body(ag_ref, af_ref, mb_ref, ng_ref, nf_ref, bh_ref, bs_ref, o_ref):
    o_ref[...] = jnp.maximum(
        ag_ref[...] * ng_ref[...] + bh_ref[...]
        + af_ref[...] * nf_ref[...] + bs_ref[...] + mb_ref[...], 0.0)


def _fuse(ag, af, mb, norm_g, norm_f, bh, bs, bm_rows):
    n, d = ag.shape
    grid = (n // bm_rows,)
    row_spec = pl.BlockSpec((bm_rows, d), lambda i: (i, 0))
    nrm_spec = pl.BlockSpec((bm_rows, 1), lambda i: (i, 0))
    b_spec = pl.BlockSpec((1, d), lambda i: (0, 0))
    return pl.pallas_call(
        _fuse_body,
        grid=grid,
        in_specs=[row_spec, row_spec, row_spec, nrm_spec, nrm_spec,
                  b_spec, b_spec],
        out_specs=row_spec,
        out_shape=jax.ShapeDtypeStruct((n, d), jnp.float32),
    )(ag, af, mb, norm_g, norm_f, bh.reshape(1, d), bs.reshape(1, d))


def kernel(h, s, m, edge_index_g, edge_index_f, norm_g, norm_f,
           wh, ws, wm, bh, bs, bm):
    n, d_in = h.shape
    d = wh.shape[1]
    e = edge_index_g.shape[1]

    bm_rows = 2000 if n % 2000 == 0 else 400

    hw, sw, mb = _mm3(h, s, m, wh, ws, wm, norm_g, norm_f, bm, bm_rows)

    blk = 3 * _C
    ept = -(-e // (_NS * blk)) * blk
    epad = ept * _NS
    npad = -(-(n + 1) // (_NS * 8)) * (_NS * 8)
    pad = epad - e
    src_g = edge_index_g[0]
    dst_g = edge_index_g[1]
    src_f = edge_index_f[0]
    dst_f = edge_index_f[1]
    if pad:
        zpad = jnp.zeros((pad,), jnp.int32)
        dpad = jnp.full((pad,), n, jnp.int32)
        src_g = jnp.concatenate([src_g, zpad])
        dst_g = jnp.concatenate([dst_g, dpad])
        src_f = jnp.concatenate([src_f, zpad])
        dst_f = jnp.concatenate([dst_f, dpad])
    zeros = jnp.zeros((npad, d), jnp.float32)

    agg_g, agg_f = _sc_aggregate(hw, sw, src_g, dst_g, src_f, dst_f,
                                 zeros, npad, ept)

    return _fuse(agg_g[:n], agg_f[:n], mb, norm_g, norm_f, bh, bs, bm_rows)


# speedup vs baseline: 1.1246x; 1.1246x over previous
import functools

import jax
import jax.numpy as jnp
from jax import lax
from jax.experimental import pallas as pl
from jax.experimental.pallas import tpu as pltpu
from jax.experimental.pallas import tpu_sc as plsc

_NS = 16
_NC = 2
_C = 256


def _mm3_body(h_ref, s_ref, m_ref, wh_ref, ws_ref, wm_ref, ng_ref, nf_ref,
              bm_ref, hw_ref, sw_ref, mb_ref):
    hw_ref[...] = jnp.dot(h_ref[...], wh_ref[...],
                          preferred_element_type=jnp.float32) * ng_ref[...]
    sw_ref[...] = jnp.dot(s_ref[...], ws_ref[...],
                          preferred_element_type=jnp.float32) * nf_ref[...]
    mb_ref[...] = jnp.dot(m_ref[...], wm_ref[...],
                          preferred_element_type=jnp.float32) + bm_ref[...]


def _mm3(h, s, m, wh, ws, wm, norm_g, norm_f, bm, bm_rows):
    n, d_in = h.shape
    d_out = wh.shape[1]
    grid = (n // bm_rows,)
    row_spec = pl.BlockSpec((bm_rows, d_in), lambda i: (i, 0))
    out_spec = pl.BlockSpec((bm_rows, d_out), lambda i: (i, 0))
    w_spec = pl.BlockSpec(wh.shape, lambda i: (0, 0))
    nrm_spec = pl.BlockSpec((bm_rows, 1), lambda i: (i, 0))
    b_spec = pl.BlockSpec((1, d_out), lambda i: (0, 0))
    out_shape = jax.ShapeDtypeStruct((n, d_out), jnp.float32)
    return pl.pallas_call(
        _mm3_body,
        grid=grid,
        in_specs=[row_spec, row_spec, row_spec, w_spec, w_spec, w_spec,
                  nrm_spec, nrm_spec, b_spec],
        out_specs=[out_spec, out_spec, out_spec],
        out_shape=[out_shape, out_shape, out_shape],
    )(h, s, m, wh, ws, wm, norm_g, norm_f, bm.reshape(1, d_out))


def _sc_aggregate(hw, sw, srcg, dstg, srcf, dstf, zeros, npad, ept):
    d = hw.shape[1]
    rows_pt = npad // _NS
    mesh = plsc.VectorSubcoreMesh(core_axis_name="c", subcore_axis_name="s")
    out_t = jax.ShapeDtypeStruct((npad, d), jnp.float32)

    @functools.partial(
        pl.kernel,
        out_type=[out_t, out_t],
        mesh=mesh,
        scratch_types=[
            pltpu.VMEM((_C,), jnp.int32),
            pltpu.VMEM((_C,), jnp.int32),
            pltpu.VMEM((_C, d), jnp.float32),
            pltpu.VMEM_SHARED((npad, d), jnp.float32),
            pltpu.SemaphoreType.DMA,
        ],
    )
    def agg(hw_h, sw_h, srcg_h, dstg_h, srcf_h, dstf_h, zeros_h,
            outg_h, outf_h, idx_s, idx_d, rows, acc, sem):
        c = lax.axis_index("c")
        s = lax.axis_index("s")
        r0 = s * rows_pt
        pltpu.sync_copy(zeros_h.at[pl.ds(r0, rows_pt)],
                        acc.at[pl.ds(r0, rows_pt)])
        plsc.subcore_barrier()

        ebase = s * ept

        def edge_loop(table_h, src_h, dst_h):
            def body(i, carry):
                base = ebase + i * _C
                pltpu.sync_copy(src_h.at[pl.ds(base, _C)], idx_s)
                pltpu.sync_copy(dst_h.at[pl.ds(base, _C)], idx_d)
                pltpu.async_copy(table_h.at[idx_s], rows, sem).wait()
                pltpu.sync_copy(rows, acc.at[idx_d], add=True)
                return carry
            lax.fori_loop(0, ept // _C, body, 0)

        @pl.when(c == 0)
        def _():
            edge_loop(hw_h, srcg_h, dstg_h)

        @pl.when(c == 1)
        def _():
            edge_loop(sw_h, srcf_h, dstf_h)

        plsc.subcore_barrier()

        @pl.when(c == 0)
        def _():
            pltpu.sync_copy(acc.at[pl.ds(r0, rows_pt)],
                            outg_h.at[pl.ds(r0, rows_pt)])

        @pl.when(c == 1)
        def _():
            pltpu.sync_copy(acc.at[pl.ds(r0, rows_pt)],
                            outf_h.at[pl.ds(r0, rows_pt)])

    return agg(hw, sw, srcg, dstg, srcf, dstf, zeros)


def _fuse_body(ag_ref, af_ref, mb_ref, ng_ref, nf_ref, bh_ref, bs_ref, o_ref):
    o_ref[...] = jnp.maximum(
        ag_ref[...] * ng_ref[...] + bh_ref[...]
        + af_ref[...] * nf_ref[...] + bs_ref[...] + mb_ref[...], 0.0)


def _fuse(ag, af, mb, norm_g, norm_f, bh, bs, bm_rows):
    n, d = ag.shape
    grid = (n // bm_rows,)
    row_spec = pl.BlockSpec((bm_rows, d), lambda i: (i, 0))
    nrm_spec = pl.BlockSpec((bm_rows, 1), lambda i: (i, 0))
    b_spec = pl.BlockSpec((1, d), lambda i: (0, 0))
    return pl.pallas_call(
        _fuse_body,
        grid=grid,
        in_specs=[row_spec, row_spec, row_spec, nrm_spec, nrm_spec,
                  b_spec, b_spec],
        out_specs=row_spec,
        out_shape=jax.ShapeDtypeStruct((n, d), jnp.float32),
    )(ag, af, mb, norm_g, norm_f, bh.reshape(1, d), bs.reshape(1, d))


def kernel(h, s, m, edge_index_g, edge_index_f, norm_g, norm_f,
           wh, ws, wm, bh, bs, bm):
    n, d_in = h.shape
    d = wh.shape[1]
    e = edge_index_g.shape[1]

    bm_rows = 2000 if n % 2000 == 0 else 400

    hw, sw, mb = _mm3(h, s, m, wh, ws, wm, norm_g, norm_f, bm, bm_rows)

    blk = _C
    ept = -(-e // (_NS * blk)) * blk
    epad = ept * _NS
    npad = -(-(n + 1) // (_NS * 8)) * (_NS * 8)
    pad = epad - e
    src_g = edge_index_g[0]
    dst_g = edge_index_g[1]
    src_f = edge_index_f[0]
    dst_f = edge_index_f[1]
    if pad:
        zpad = jnp.zeros((pad,), jnp.int32)
        dpad = jnp.full((pad,), n, jnp.int32)
        src_g = jnp.concatenate([src_g, zpad])
        dst_g = jnp.concatenate([dst_g, dpad])
        src_f = jnp.concatenate([src_f, zpad])
        dst_f = jnp.concatenate([dst_f, dpad])
    zeros = jnp.zeros((npad, d), jnp.float32)

    agg_g, agg_f = _sc_aggregate(hw, sw, src_g, dst_g, src_f, dst_f,
                                 zeros, npad, ept)

    return _fuse(agg_g[:n], agg_f[:n], mb, norm_g, norm_f, bh, bs, bm_rows)


# speedup vs baseline: 1.1410x; 1.0146x over previous
import functools

import jax
import jax.numpy as jnp
from jax import lax
from jax.experimental import pallas as pl
from jax.experimental.pallas import tpu as pltpu
from jax.experimental.pallas import tpu_sc as plsc

_NS = 16
_NC = 2
_C = 256


def _scale2_body(h_ref, s_ref, ng_ref, nf_ref, hn_ref, sn_ref):
    hn_ref[...] = h_ref[...] * ng_ref[...]
    sn_ref[...] = s_ref[...] * nf_ref[...]


def _scale2(h, s, norm_g, norm_f, bm_rows):
    n, d = h.shape
    grid = (n // bm_rows,)
    row_spec = pl.BlockSpec((bm_rows, d), lambda i: (i, 0))
    nrm_spec = pl.BlockSpec((bm_rows, 1), lambda i: (i, 0))
    out_shape = jax.ShapeDtypeStruct((n, d), jnp.float32)
    return pl.pallas_call(
        _scale2_body,
        grid=grid,
        in_specs=[row_spec, row_spec, nrm_spec, nrm_spec],
        out_specs=[row_spec, row_spec],
        out_shape=[out_shape, out_shape],
    )(h, s, norm_g, norm_f)


def _sc_aggregate(hn, sn, srcg, dstg, srcf, dstf, zeros, npad, ept):
    d = hn.shape[1]
    rows_pt = npad // _NS
    mesh = plsc.VectorSubcoreMesh(core_axis_name="c", subcore_axis_name="s")
    out_t = jax.ShapeDtypeStruct((npad, d), jnp.float32)

    @functools.partial(
        pl.kernel,
        out_type=[out_t, out_t],
        mesh=mesh,
        scratch_types=[
            pltpu.VMEM((_C,), jnp.int32),
            pltpu.VMEM((_C,), jnp.int32),
            pltpu.VMEM((_C, d), jnp.float32),
            pltpu.VMEM_SHARED((npad, d), jnp.float32),
            pltpu.SemaphoreType.DMA,
        ],
    )
    def agg(hn_h, sn_h, srcg_h, dstg_h, srcf_h, dstf_h, zeros_h,
            outg_h, outf_h, idx_s, idx_d, rows, acc, sem):
        c = lax.axis_index("c")
        s = lax.axis_index("s")
        r0 = s * rows_pt
        pltpu.sync_copy(zeros_h.at[pl.ds(r0, rows_pt)],
                        acc.at[pl.ds(r0, rows_pt)])
        plsc.subcore_barrier()

        ebase = s * ept

        def edge_loop(table_h, src_h, dst_h):
            def body(i, carry):
                base = ebase + i * _C
                pltpu.sync_copy(src_h.at[pl.ds(base, _C)], idx_s)
                pltpu.sync_copy(dst_h.at[pl.ds(base, _C)], idx_d)
                pltpu.async_copy(table_h.at[idx_s], rows, sem).wait()
                pltpu.sync_copy(rows, acc.at[idx_d], add=True)
                return carry
            lax.fori_loop(0, ept // _C, body, 0)

        @pl.when(c == 0)
        def _():
            edge_loop(hn_h, srcg_h, dstg_h)

        @pl.when(c == 1)
        def _():
            edge_loop(sn_h, srcf_h, dstf_h)

        plsc.subcore_barrier()

        @pl.when(c == 0)
        def _():
            pltpu.sync_copy(acc.at[pl.ds(r0, rows_pt)],
                            outg_h.at[pl.ds(r0, rows_pt)])

        @pl.when(c == 1)
        def _():
            pltpu.sync_copy(acc.at[pl.ds(r0, rows_pt)],
                            outf_h.at[pl.ds(r0, rows_pt)])

    return agg(hn, sn, srcg, dstg, srcf, dstf, zeros)


def _fuse3_body(ag_ref, af_ref, m_ref, wh_ref, ws_ref, wm_ref,
                ng_ref, nf_ref, bias_ref, o_ref):
    hg = jnp.dot(ag_ref[...], wh_ref[...],
                 preferred_element_type=jnp.float32) * ng_ref[...]
    hf = jnp.dot(af_ref[...], ws_ref[...],
                 preferred_element_type=jnp.float32) * nf_ref[...]
    hm = jnp.dot(m_ref[...], wm_ref[...],
                 preferred_element_type=jnp.float32)
    o_ref[...] = jnp.maximum(hg + hf + hm + bias_ref[...], 0.0)


def _fuse3(ag, af, m, wh, ws, wm, norm_g, norm_f, bias, bm_rows):
    n, d = m.shape
    d_out = wh.shape[1]
    grid = (n // bm_rows,)
    row_spec = pl.BlockSpec((bm_rows, d), lambda i: (i, 0))
    out_spec = pl.BlockSpec((bm_rows, d_out), lambda i: (i, 0))
    w_spec = pl.BlockSpec((d, d_out), lambda i: (0, 0))
    nrm_spec = pl.BlockSpec((bm_rows, 1), lambda i: (i, 0))
    b_spec = pl.BlockSpec((1, d_out), lambda i: (0, 0))
    return pl.pallas_call(
        _fuse3_body,
        grid=grid,
        in_specs=[row_spec, row_spec, row_spec, w_spec, w_spec, w_spec,
                  nrm_spec, nrm_spec, b_spec],
        out_specs=out_spec,
        out_shape=jax.ShapeDtypeStruct((n, d_out), jnp.float32),
    )(ag, af, m, wh, ws, wm, norm_g, norm_f, bias)


def kernel(h, s, m, edge_index_g, edge_index_f, norm_g, norm_f,
           wh, ws, wm, bh, bs, bm):
    n, d = h.shape
    e = edge_index_g.shape[1]

    bm_rows = 2000 if n % 2000 == 0 else 400

    hn, sn = _scale2(h, s, norm_g, norm_f, bm_rows)

    ept = -(-e // (_NS * _C)) * _C
    epad = ept * _NS
    npad = -(-(n + 1) // (_NS * 8)) * (_NS * 8)
    pad = epad - e
    src_g = edge_index_g[0]
    dst_g = edge_index_g[1]
    src_f = edge_index_f[0]
    dst_f = edge_index_f[1]
    if pad:
        zpad = jnp.zeros((pad,), jnp.int32)
        dpad = jnp.full((pad,), n, jnp.int32)
        src_g = jnp.concatenate([src_g, zpad])
        dst_g = jnp.concatenate([dst_g, dpad])
        src_f = jnp.concatenate([src_f, zpad])
        dst_f = jnp.concatenate([dst_f, dpad])
    zeros = jnp.zeros((npad, d), jnp.float32)

    agg_g, agg_f = _sc_aggregate(hn, sn, src_g, dst_g, src_f, dst_f,
                                 zeros, npad, ept)

    bias = (bh + bs + bm).reshape(1, wh.shape[1])
    return _fuse3(agg_g[:n], agg_f[:n], m, wh, ws, wm, norm_g, norm_f,
                  bias, bm_rows)


# speedup vs baseline: 1.1867x; 1.0401x over previous
import functools

import jax
import jax.numpy as jnp
from jax import lax
from jax.experimental import pallas as pl
from jax.experimental.pallas import tpu as pltpu
from jax.experimental.pallas import tpu_sc as plsc

_NS = 16
_NC = 2
_C = 256


def _scale2_body(h_ref, s_ref, ng_ref, nf_ref, hn_ref, sn_ref):
    hn_ref[...] = h_ref[...] * ng_ref[...]
    sn_ref[...] = s_ref[...] * nf_ref[...]


def _scale2(h, s, norm_g, norm_f, bm_rows):
    n, d = h.shape
    grid = (n // bm_rows,)
    row_spec = pl.BlockSpec((bm_rows, d), lambda i: (i, 0))
    nrm_spec = pl.BlockSpec((bm_rows, 1), lambda i: (i, 0))
    out_shape = jax.ShapeDtypeStruct((n, d), jnp.float32)
    return pl.pallas_call(
        _scale2_body,
        grid=grid,
        in_specs=[row_spec, row_spec, nrm_spec, nrm_spec],
        out_specs=[row_spec, row_spec],
        out_shape=[out_shape, out_shape],
    )(h, s, norm_g, norm_f)


def _sc_aggregate(hn, sn, edges_g, edges_f, zeros, npad, ept):
    d = hn.shape[1]
    rows_pt = npad // _NS
    mesh = plsc.VectorSubcoreMesh(core_axis_name="c", subcore_axis_name="s")
    out_t = jax.ShapeDtypeStruct((npad, d), jnp.float32)

    @functools.partial(
        pl.kernel,
        out_type=[out_t, out_t],
        mesh=mesh,
        scratch_types=[
            pltpu.VMEM((2 * _C,), jnp.int32),
            pltpu.VMEM((_C, d), jnp.float32),
            pltpu.VMEM_SHARED((npad, d), jnp.float32),
            pltpu.SemaphoreType.DMA,
        ],
    )
    def agg(hn_h, sn_h, edges_g_h, edges_f_h, zeros_h,
            outg_h, outf_h, idx, rows, acc, sem):
        c = lax.axis_index("c")
        s = lax.axis_index("s")
        r0 = s * rows_pt
        pltpu.sync_copy(zeros_h.at[pl.ds(r0, rows_pt)],
                        acc.at[pl.ds(r0, rows_pt)])
        plsc.subcore_barrier()

        ebase = s * ept

        ebase2 = 2 * ebase

        def edge_loop(table_h, edges_h):
            def body(i, carry):
                base2 = ebase2 + i * (2 * _C)
                pltpu.sync_copy(edges_h.at[pl.ds(base2, 2 * _C)], idx)
                pltpu.async_copy(table_h.at[idx.at[pl.ds(0, _C)]], rows,
                                 sem).wait()
                pltpu.sync_copy(rows, acc.at[idx.at[pl.ds(_C, _C)]],
                                add=True)
                return carry
            lax.fori_loop(0, ept // _C, body, 0)

        @pl.when(c == 0)
        def _():
            edge_loop(hn_h, edges_g_h)

        @pl.when(c == 1)
        def _():
            edge_loop(sn_h, edges_f_h)

        plsc.subcore_barrier()

        @pl.when(c == 0)
        def _():
            pltpu.sync_copy(acc.at[pl.ds(r0, rows_pt)],
                            outg_h.at[pl.ds(r0, rows_pt)])

        @pl.when(c == 1)
        def _():
            pltpu.sync_copy(acc.at[pl.ds(r0, rows_pt)],
                            outf_h.at[pl.ds(r0, rows_pt)])

    return agg(hn, sn, edges_g, edges_f, zeros)


def _fuse3_body(ag_ref, af_ref, m_ref, wh_ref, ws_ref, wm_ref,
                ng_ref, nf_ref, bias_ref, o_ref):
    hg = jnp.dot(ag_ref[...], wh_ref[...],
                 preferred_element_type=jnp.float32) * ng_ref[...]
    hf = jnp.dot(af_ref[...], ws_ref[...],
                 preferred_element_type=jnp.float32) * nf_ref[...]
    hm = jnp.dot(m_ref[...], wm_ref[...],
                 preferred_element_type=jnp.float32)
    o_ref[...] = jnp.maximum(hg + hf + hm + bias_ref[...], 0.0)


def _fuse3(ag, af, m, wh, ws, wm, norm_g, norm_f, bias, bm_rows):
    n, d = m.shape
    d_out = wh.shape[1]
    grid = (n // bm_rows,)
    row_spec = pl.BlockSpec((bm_rows, d), lambda i: (i, 0))
    out_spec = pl.BlockSpec((bm_rows, d_out), lambda i: (i, 0))
    w_spec = pl.BlockSpec((d, d_out), lambda i: (0, 0))
    nrm_spec = pl.BlockSpec((bm_rows, 1), lambda i: (i, 0))
    b_spec = pl.BlockSpec((1, d_out), lambda i: (0, 0))
    return pl.pallas_call(
        _fuse3_body,
        grid=grid,
        in_specs=[row_spec, row_spec, row_spec, w_spec, w_spec, w_spec,
                  nrm_spec, nrm_spec, b_spec],
        out_specs=out_spec,
        out_shape=jax.ShapeDtypeStruct((n, d_out), jnp.float32),
    )(ag, af, m, wh, ws, wm, norm_g, norm_f, bias)


def kernel(h, s, m, edge_index_g, edge_index_f, norm_g, norm_f,
           wh, ws, wm, bh, bs, bm):
    n, d = h.shape
    e = edge_index_g.shape[1]

    bm_rows = 2000 if n % 2000 == 0 else 400

    hn, sn = _scale2(h, s, norm_g, norm_f, bm_rows)

    ept = -(-e // (_NS * _C)) * _C
    epad = ept * _NS
    npad = -(-(n + 1) // (_NS * 8)) * (_NS * 8)
    pad = epad - e
    src_g = edge_index_g[0]
    dst_g = edge_index_g[1]
    src_f = edge_index_f[0]
    dst_f = edge_index_f[1]
    if pad:
        zpad = jnp.zeros((pad,), jnp.int32)
        dpad = jnp.full((pad,), n, jnp.int32)
        src_g = jnp.concatenate([src_g, zpad])
        dst_g = jnp.concatenate([dst_g, dpad])
        src_f = jnp.concatenate([src_f, zpad])
        dst_f = jnp.concatenate([dst_f, dpad])
    def _inter(src, dst):
        return jnp.stack([src.reshape(-1, _C), dst.reshape(-1, _C)],
                         axis=1).reshape(-1)
    edges_g = _inter(src_g, dst_g)
    edges_f = _inter(src_f, dst_f)
    zeros = jnp.zeros((npad, d), jnp.float32)

    agg_g, agg_f = _sc_aggregate(hn, sn, edges_g, edges_f,
                                 zeros, npad, ept)

    bias = (bh + bs + bm).reshape(1, wh.shape[1])
    return _fuse3(agg_g[:n], agg_f[:n], m, wh, ws, wm, norm_g, norm_f,
                  bias, bm_rows)
